# Initial kernel scaffold; baseline (speedup 1.0000x reference)
import jax
import jax.numpy as jnp
from jax.experimental import pallas as pl


def kernel(x, edge_index, edge_type, edge_attr, W1, a_src1, a_dst1, b1, W2, a_src2, a_dst2, b2, Wl, bl):
    raise NotImplementedError("write your pallas kernel here")


# speedup vs baseline: 1.1470x; 1.1470x over previous
import jax
import jax.numpy as jnp
from jax.experimental import pallas as pl

_N = 10000
_H = 8
_C = 128


def _mm_kernel(x_ref, w_ref, o_ref):
    o_ref[...] = jnp.dot(x_ref[...], w_ref[...], preferred_element_type=jnp.float32)


def _matmul(x, w, bn=1000):
    n, k = x.shape
    m = w.shape[1]
    return pl.pallas_call(
        _mm_kernel,
        grid=(n // bn,),
        in_specs=[
            pl.BlockSpec((bn, k), lambda i: (i, 0)),
            pl.BlockSpec((k, m), lambda i: (0, 0)),
        ],
        out_specs=pl.BlockSpec((bn, m), lambda i: (i, 0)),
        out_shape=jax.ShapeDtypeStruct((n, m), jnp.float32),
    )(x, w)


def kernel(x, edge_index, edge_type, edge_attr, W1, a_src1, a_dst1, b1, W2, a_src2, a_dst2, b2, Wl, bl):
    N = x.shape[0]
    src0 = edge_index[0].astype(jnp.int32)
    dst0 = edge_index[1].astype(jnp.int32)
    loop = jnp.arange(N, dtype=jnp.int32)
    src = jnp.concatenate([src0, loop])
    dst = jnp.concatenate([dst0, loop])

    h1 = _matmul(x, W1)
    hr = h1.reshape(N, _H, _C)
    AS = jnp.sum(hr * a_src1[None], -1)
    AD = jnp.sum(hr * a_dst1[None], -1)
    M1 = jax.nn.leaky_relu(AS.max(0) + AD.max(0), 0.2)
    e = jax.nn.leaky_relu(AS[src] + AD[dst], 0.2)
    ee = jnp.exp(e - M1[None])
    denom = jax.ops.segment_sum(ee, dst, num_segments=N)
    U = jax.ops.segment_sum(hr[src] * ee[:, :, None], dst, num_segments=N)
    out1 = (U / denom[:, :, None]).reshape(N, _H * _C) + b1
    h1a = jax.nn.leaky_relu(out1, 0.01)

    h2 = _matmul(h1a, W2)
    AS2 = h2 @ a_src2[0]
    AD2 = h2 @ a_dst2[0]
    M2 = jax.nn.leaky_relu(AS2.max() + AD2.max(), 0.2)
    e2 = jax.nn.leaky_relu(AS2[src] + AD2[dst], 0.2)
    ee2 = jnp.exp(e2 - M2)
    den2 = jax.ops.segment_sum(ee2, dst, num_segments=N)
    a2 = ee2 / den2[dst]
    w = jax.ops.segment_sum(a2, src, num_segments=N)
    pooled = w @ h2 + N * b2
    return (pooled @ Wl + bl)[None, :]
